# baseline (speedup 1.0000x reference)
import functools

import jax
import jax.numpy as jnp
from jax import lax
from jax.experimental import pallas as pl
from jax.experimental.pallas import tpu as pltpu
from jax.experimental.pallas import tpu_sc as plsc

_NC = 2
_NS = 16
_NW = _NC * _NS
_SCHED = (8, 24, 64, 80, 80)
_BUFROWS = 80
_NBUF = 2


@functools.partial(jax.jit, static_argnums=(0, 1, 2))
def _build_and_run(batch, seq_len, embed, table, pos):
    mesh = plsc.VectorSubcoreMesh(core_axis_name="c", subcore_axis_name="s")
    nchunk = len(_SCHED)
    offs = [sum(_SCHED[:i]) for i in range(nchunk)]
    rpw = sum(_SCHED)

    @functools.partial(
        pl.kernel,
        out_type=jax.ShapeDtypeStruct((batch * seq_len, embed), jnp.float32),
        mesh=mesh,
        scratch_types=(
            [pltpu.VMEM((rpw,), jnp.int32)]
            + [pltpu.VMEM((_BUFROWS, embed), jnp.float32)] * _NBUF
            + [pltpu.SemaphoreType.DMA] * (2 * _NBUF)
        ),
    )
    def pos_embed(table_hbm, pos_hbm, out_hbm, idx_v, *rest):
        bufs = rest[:_NBUF]
        gsems = rest[_NBUF:2 * _NBUF]
        wsems = rest[2 * _NBUF:]
        wid = lax.axis_index("s") * _NC + lax.axis_index("c")
        base = wid * rpw
        pltpu.sync_copy(pos_hbm.at[wid], idx_v)

        def gather(i):
            rows = _SCHED[i]
            sl = i % _NBUF
            return pltpu.async_copy(
                table_hbm.at[idx_v.at[pl.ds(offs[i], rows)]],
                bufs[sl].at[pl.ds(0, rows)], gsems[sl])

        gh = [None] * nchunk
        wh = [[] for _ in range(nchunk)]
        for j in range(_NBUF - 1):
            gh[j] = gather(j)
        for i in range(nchunk):
            rows = _SCHED[i]
            sl = i % _NBUF
            if i >= 1:
                for h in wh[i - 1]:
                    h.wait()
            if i + 1 < nchunk:
                gh[i + 1] = gather(i + 1)
            gh[i].wait()
            for b in range(batch):
                wh[i].append(pltpu.async_copy(
                    bufs[sl].at[pl.ds(0, rows)],
                    out_hbm.at[pl.ds(b * seq_len + base + offs[i], rows)],
                    wsems[sl]))
        for h in wh[nchunk - 1]:
            h.wait()

    return pos_embed(table, pos)


def kernel(inputs, length, table):
    batch, seq_len = inputs.shape
    vocab, embed = table.shape
    pos = jnp.clip(
        jnp.arange(seq_len, dtype=jnp.int32) + jnp.asarray(length, jnp.int32),
        0, vocab - 1)
    pos = pos.reshape(_NW, seq_len // _NW)
    out = _build_and_run(batch, seq_len, embed, table, pos)
    return out.reshape(batch, seq_len, embed)


# speedup vs baseline: 1.0112x; 1.0112x over previous
import functools

import jax
import jax.numpy as jnp
from jax import lax
from jax.experimental import pallas as pl
from jax.experimental.pallas import tpu as pltpu
from jax.experimental.pallas import tpu_sc as plsc

_NC = 2
_NS = 16
_NW = _NC * _NS
_SCHED = (16, 80, 80, 80)
_BUFROWS = 80
_NBUF = 2


@functools.partial(jax.jit, static_argnums=(0, 1, 2))
def _build_and_run(batch, seq_len, embed, table, pos):
    mesh = plsc.VectorSubcoreMesh(core_axis_name="c", subcore_axis_name="s")
    nchunk = len(_SCHED)
    offs = [sum(_SCHED[:i]) for i in range(nchunk)]
    rpw = sum(_SCHED)

    @functools.partial(
        pl.kernel,
        out_type=jax.ShapeDtypeStruct((batch * seq_len, embed), jnp.float32),
        mesh=mesh,
        scratch_types=(
            [pltpu.VMEM((rpw,), jnp.int32)]
            + [pltpu.VMEM((_BUFROWS, embed), jnp.float32)] * _NBUF
            + [pltpu.SemaphoreType.DMA] * (2 * _NBUF)
        ),
    )
    def pos_embed(table_hbm, pos_hbm, out_hbm, idx_v, *rest):
        bufs = rest[:_NBUF]
        gsems = rest[_NBUF:2 * _NBUF]
        wsems = rest[2 * _NBUF:]
        wid = lax.axis_index("s") * _NC + lax.axis_index("c")
        base = wid * rpw
        pltpu.sync_copy(pos_hbm.at[wid], idx_v)

        def gather(i):
            rows = _SCHED[i]
            sl = i % _NBUF
            return pltpu.async_copy(
                table_hbm.at[idx_v.at[pl.ds(offs[i], rows)]],
                bufs[sl].at[pl.ds(0, rows)], gsems[sl])

        gh = [None] * nchunk
        wh = [[] for _ in range(nchunk)]
        for j in range(_NBUF - 1):
            gh[j] = gather(j)
        for i in range(nchunk):
            rows = _SCHED[i]
            sl = i % _NBUF
            if i >= 1:
                for h in wh[i - 1]:
                    h.wait()
            if i + 1 < nchunk:
                gh[i + 1] = gather(i + 1)
            gh[i].wait()
            for b in range(batch):
                wh[i].append(pltpu.async_copy(
                    bufs[sl].at[pl.ds(0, rows)],
                    out_hbm.at[pl.ds(b * seq_len + base + offs[i], rows)],
                    wsems[sl]))
        for h in wh[nchunk - 1]:
            h.wait()

    return pos_embed(table, pos)


def kernel(inputs, length, table):
    batch, seq_len = inputs.shape
    vocab, embed = table.shape
    pos = jnp.clip(
        jnp.arange(seq_len, dtype=jnp.int32) + jnp.asarray(length, jnp.int32),
        0, vocab - 1)
    pos = pos.reshape(_NW, seq_len // _NW)
    out = _build_and_run(batch, seq_len, embed, table, pos)
    return out.reshape(batch, seq_len, embed)
